# Initial kernel scaffold; baseline (speedup 1.0000x reference)
import jax
import jax.numpy as jnp
from jax.experimental import pallas as pl


def kernel(mu, log_sigma, indices, eps):
    raise NotImplementedError("write your pallas kernel here")


# speedup vs baseline: 1.3492x; 1.3492x over previous
import functools

import jax
import jax.numpy as jnp
from jax import lax
from jax.experimental import pallas as pl
from jax.experimental.pallas import tpu as pltpu
from jax.experimental.pallas import tpu_sc as plsc

N_SAMPLES = 8
BATCH = 16384
NC = 2
NS = 16
LANES = 16
NW = NC * NS
BPW = BATCH // NW
CHUNK = 128
NCHUNK = BPW // CHUNK


def _sc_body(mu_hbm, ls_hbm, idx_hbm, eps_hbm, out_hbm,
             idx_v, mu_v, ls_v, eps_v, out_v, sem):
    wid = lax.axis_index("s") * NC + lax.axis_index("c")
    base = wid * BPW

    pltpu.sync_copy(idx_hbm.at[wid], idx_v)

    copies = []
    for j in range(NCHUNK):
        copies.append(pltpu.async_copy(
            mu_hbm.at[idx_v.at[j]], mu_v.at[j], sem))
        copies.append(pltpu.async_copy(
            ls_hbm.at[idx_v.at[j]], ls_v.at[j], sem))
    copies.append(pltpu.async_copy(
        eps_hbm.at[:, pl.ds(base, BPW)], eps_v, sem))
    for c in copies:
        c.wait()

    for j in range(NCHUNK):
        for l in range(CHUNK // LANES):
            sl = pl.ds(l * LANES, LANES)
            m16 = mu_v[j, sl]
            s16 = jnp.exp(ls_v[j, sl])
            col = pl.ds(j * CHUNK + l * LANES, LANES)
            for s in range(N_SAMPLES):
                out_v[s, col] = m16 + s16 * eps_v[s, col]

    pltpu.sync_copy(out_v, out_hbm.at[:, pl.ds(base, BPW)])


@jax.jit
def kernel(mu, log_sigma, indices, eps):
    idx = indices.astype(jnp.int32).reshape(NW, NCHUNK, CHUNK)
    mesh = plsc.VectorSubcoreMesh(
        core_axis_name="c", subcore_axis_name="s",
        num_cores=NC, num_subcores=NS)
    run = pl.kernel(
        _sc_body,
        out_type=jax.ShapeDtypeStruct((N_SAMPLES, BATCH), jnp.float32),
        mesh=mesh,
        scratch_types=[
            pltpu.VMEM((NCHUNK, CHUNK), jnp.int32),
            pltpu.VMEM((NCHUNK, CHUNK), jnp.float32),
            pltpu.VMEM((NCHUNK, CHUNK), jnp.float32),
            pltpu.VMEM((N_SAMPLES, BPW), jnp.float32),
            pltpu.VMEM((N_SAMPLES, BPW), jnp.float32),
            pltpu.SemaphoreType.DMA,
        ],
    )
    return run(mu, log_sigma, idx, eps)
